# baseline (speedup 1.0000x reference)
import functools

import jax
import jax.numpy as jnp
from jax.experimental import pallas as pl
from jax.experimental.pallas import tpu as pltpu

_HTILE = 256


def _router_block_kernel(x_ref, w1_ref, b1_ref, w2_ref, b2_ref, out_ref,
                         w1bf_scr, logits_scr):
    i = pl.program_id(0)

    @pl.when(i == 0)
    def _():
        w1bf_scr[:] = w1_ref[:].astype(jnp.bfloat16)

    x_bf16 = x_ref[:].astype(jnp.bfloat16)
    d_hidden = w1_ref.shape[1]
    acc = None
    for j in range(0, d_hidden, _HTILE):
        hj = jnp.dot(x_bf16, w1bf_scr[:, j:j + _HTILE],
                     preferred_element_type=jnp.float32)
        hj = jnp.maximum(hj + b1_ref[:, j:j + _HTILE], 0.0)
        part = jnp.dot(hj.astype(jnp.bfloat16),
                       w2_ref[j:j + _HTILE, :].astype(jnp.bfloat16),
                       preferred_element_type=jnp.float32)
        acc = part if acc is None else acc + part
    new_logits = acc + b2_ref[:]

    logits = logits_scr[(i + 1) % 2]
    n_exp = logits.shape[1]
    col = jax.lax.broadcasted_iota(jnp.int32, logits.shape, 1)
    m1 = jnp.max(logits, axis=1, keepdims=True)
    i1 = jnp.min(jnp.where(logits >= m1, col, n_exp), axis=1, keepdims=True)
    l_rest = jnp.where(col == i1, -jnp.inf, logits)
    m2 = jnp.max(l_rest, axis=1, keepdims=True)
    i2 = jnp.min(jnp.where(l_rest >= m2, col, n_exp), axis=1, keepdims=True)
    t = jnp.exp(m2 - m1)
    c1 = 1.0 / (1.0 + t)
    c2 = t / (1.0 + t)
    out_ref[:] = jnp.where(col == i1, c1, jnp.where(col == i2, c2, 0.0))

    logits_scr[i % 2] = new_logits


@functools.partial(jax.jit, static_argnames=())
def kernel(x, W1, b1, W2, b2):
    n_tokens, d_in = x.shape
    d_hidden = W1.shape[1]
    n_experts = W2.shape[1]
    bm = 512
    nblk = n_tokens // bm
    grid = (nblk + 1,)

    b1_2d = b1.reshape(1, d_hidden)
    b2_2d = b2.reshape(1, n_experts)

    return pl.pallas_call(
        _router_block_kernel,
        grid=grid,
        in_specs=[
            pl.BlockSpec((bm, d_in), lambda i: (jnp.minimum(i, nblk - 1), 0)),
            pl.BlockSpec((d_in, d_hidden), lambda i: (0, 0)),
            pl.BlockSpec((1, d_hidden), lambda i: (0, 0)),
            pl.BlockSpec((d_hidden, n_experts), lambda i: (0, 0)),
            pl.BlockSpec((1, n_experts), lambda i: (0, 0)),
        ],
        out_specs=pl.BlockSpec((bm, n_experts),
                               lambda i: (jnp.maximum(i - 1, 0), 0)),
        out_shape=jax.ShapeDtypeStruct((n_tokens, n_experts), jnp.float32),
        scratch_shapes=[
            pltpu.VMEM((d_in, d_hidden), jnp.bfloat16),
            pltpu.VMEM((2, bm, n_experts), jnp.float32),
        ],
        compiler_params=pltpu.CompilerParams(
            dimension_semantics=("arbitrary",),
        ),
    )(x, W1, b1_2d, W2, b2_2d)


# speedup vs baseline: 1.6088x; 1.6088x over previous
import functools

import jax
import jax.numpy as jnp
from jax.experimental import pallas as pl
from jax.experimental.pallas import tpu as pltpu


def _route_top2(logits):
    n_exp = logits.shape[1]
    col = jax.lax.broadcasted_iota(jnp.int32, logits.shape, 1)
    m1 = jnp.max(logits, axis=1, keepdims=True)
    i1 = jnp.min(jnp.where(logits >= m1, col, n_exp), axis=1, keepdims=True)
    l_rest = jnp.where(col == i1, -jnp.inf, logits)
    m2 = jnp.max(l_rest, axis=1, keepdims=True)
    i2 = jnp.min(jnp.where(l_rest >= m2, col, n_exp), axis=1, keepdims=True)
    t = jnp.exp(m2 - m1)
    c1 = 1.0 / (1.0 + t)
    c2 = t / (1.0 + t)
    return jnp.where(col == i1, c1, jnp.where(col == i2, c2, 0.0))


def _router_block_kernel(x_ref, w1_ref, b1_ref, w2_ref, b2_ref,
                         out_ref, tail_ref, w1bf_scr, logits_scr):
    i = pl.program_id(0)
    nsteps = pl.num_programs(0)

    @pl.when(i == 0)
    def _():
        w1bf_scr[:] = w1_ref[:].astype(jnp.bfloat16)

    x_bf16 = x_ref[:].astype(jnp.bfloat16)
    h = jnp.dot(x_bf16, w1bf_scr[:], preferred_element_type=jnp.float32)
    h = jnp.maximum(h + b1_ref[:], 0.0)
    new_logits = jnp.dot(h.astype(jnp.bfloat16),
                         w2_ref[:].astype(jnp.bfloat16),
                         preferred_element_type=jnp.float32) + b2_ref[:]

    out_ref[:] = _route_top2(logits_scr[(i + 1) % 2])
    logits_scr[i % 2] = new_logits

    @pl.when(i == nsteps - 1)
    def _():
        tail_ref[:] = _route_top2(new_logits)


@functools.partial(jax.jit, static_argnames=())
def kernel(x, W1, b1, W2, b2):
    n_tokens, d_in = x.shape
    d_hidden = W1.shape[1]
    n_experts = W2.shape[1]
    bm = 512
    nblk = n_tokens // bm
    grid = (nblk,)

    b1_2d = b1.reshape(1, d_hidden)
    b2_2d = b2.reshape(1, n_experts)

    main, tail = pl.pallas_call(
        _router_block_kernel,
        grid=grid,
        in_specs=[
            pl.BlockSpec((bm, d_in), lambda i: (i, 0)),
            pl.BlockSpec((d_in, d_hidden), lambda i: (0, 0)),
            pl.BlockSpec((1, d_hidden), lambda i: (0, 0)),
            pl.BlockSpec((d_hidden, n_experts), lambda i: (0, 0)),
            pl.BlockSpec((1, n_experts), lambda i: (0, 0)),
        ],
        out_specs=[
            pl.BlockSpec((bm, n_experts), lambda i: (jnp.maximum(i - 1, 0), 0)),
            pl.BlockSpec((bm, n_experts), lambda i: (0, 0)),
        ],
        out_shape=[
            jax.ShapeDtypeStruct(((nblk - 1) * bm, n_experts), jnp.float32),
            jax.ShapeDtypeStruct((bm, n_experts), jnp.float32),
        ],
        scratch_shapes=[
            pltpu.VMEM((d_in, d_hidden), jnp.bfloat16),
            pltpu.VMEM((2, bm, n_experts), jnp.float32),
        ],
        compiler_params=pltpu.CompilerParams(
            dimension_semantics=("arbitrary",),
        ),
    )(x, W1, b1_2d, W2, b2_2d)
    return jnp.concatenate([main, tail], axis=0)


# speedup vs baseline: 1.6289x; 1.0125x over previous
import functools

import jax
import jax.numpy as jnp
from jax.experimental import pallas as pl
from jax.experimental.pallas import tpu as pltpu


def _route_top2(logits):
    n_exp = logits.shape[1]
    col = jax.lax.broadcasted_iota(jnp.int32, logits.shape, 1)
    m1 = jnp.max(logits, axis=1, keepdims=True)
    i1 = jnp.min(jnp.where(logits >= m1, col, n_exp), axis=1, keepdims=True)
    l_rest = jnp.where(col == i1, -jnp.inf, logits)
    m2 = jnp.max(l_rest, axis=1, keepdims=True)
    i2 = jnp.min(jnp.where(l_rest >= m2, col, n_exp), axis=1, keepdims=True)
    t = jnp.exp(m2 - m1)
    c1 = 1.0 / (1.0 + t)
    c2 = t / (1.0 + t)
    return jnp.where(col == i1, c1, jnp.where(col == i2, c2, 0.0))


def _router_block_kernel(x_ref, w1_ref, b1_ref, w2_ref, b2_ref,
                         out_ref, tail_ref, w1bf_scr, logits_scr):
    i = pl.program_id(0)
    nsteps = pl.num_programs(0)

    @pl.when(i == 0)
    def _():
        w1bf_scr[:] = w1_ref[:].astype(jnp.bfloat16)

    x_bf16 = x_ref[:].astype(jnp.bfloat16)
    h = jnp.dot(x_bf16, w1bf_scr[:], preferred_element_type=jnp.float32)
    h = jnp.maximum(h + b1_ref[:], 0.0)
    new_logits = jnp.dot(h.astype(jnp.bfloat16),
                         w2_ref[:].astype(jnp.bfloat16),
                         preferred_element_type=jnp.float32) + b2_ref[:]

    out_ref[:] = _route_top2(logits_scr[(i + 1) % 2])
    logits_scr[i % 2] = new_logits

    @pl.when(i == nsteps - 1)
    def _():
        tail_ref[:] = _route_top2(new_logits)


@functools.partial(jax.jit, static_argnames=())
def kernel(x, W1, b1, W2, b2):
    n_tokens, d_in = x.shape
    d_hidden = W1.shape[1]
    n_experts = W2.shape[1]
    bm = 1024
    nblk = n_tokens // bm
    grid = (nblk,)

    b1_2d = b1.reshape(1, d_hidden)
    b2_2d = b2.reshape(1, n_experts)

    main, tail = pl.pallas_call(
        _router_block_kernel,
        grid=grid,
        in_specs=[
            pl.BlockSpec((bm, d_in), lambda i: (i, 0)),
            pl.BlockSpec((d_in, d_hidden), lambda i: (0, 0)),
            pl.BlockSpec((1, d_hidden), lambda i: (0, 0)),
            pl.BlockSpec((d_hidden, n_experts), lambda i: (0, 0)),
            pl.BlockSpec((1, n_experts), lambda i: (0, 0)),
        ],
        out_specs=[
            pl.BlockSpec((bm, n_experts), lambda i: (jnp.maximum(i - 1, 0), 0)),
            pl.BlockSpec((bm, n_experts), lambda i: (0, 0)),
        ],
        out_shape=[
            jax.ShapeDtypeStruct(((nblk - 1) * bm, n_experts), jnp.float32),
            jax.ShapeDtypeStruct((bm, n_experts), jnp.float32),
        ],
        scratch_shapes=[
            pltpu.VMEM((d_in, d_hidden), jnp.bfloat16),
            pltpu.VMEM((2, bm, n_experts), jnp.float32),
        ],
        compiler_params=pltpu.CompilerParams(
            dimension_semantics=("arbitrary",),
        ),
    )(x, W1, b1_2d, W2, b2_2d)
    return jnp.concatenate([main, tail], axis=0)
